# baseline (speedup 1.0000x reference)
import jax
import jax.numpy as jnp
from jax.experimental import pallas as pl
from jax.experimental.pallas import tpu as pltpu


def _gather_direct_kernel(idx_ref, lut_ref, out_ref, sem):
    B = out_ref.shape[0]
    for b in range(B):
        pltpu.make_async_copy(
            lut_ref.at[pl.ds(idx_ref[b], 1), :],
            out_ref.at[pl.ds(b, 1), :],
            sem,
        ).start()
    pltpu.make_async_copy(
        lut_ref.at[pl.ds(0, B), :],
        out_ref.at[pl.ds(0, B), :],
        sem,
    ).wait()


def kernel(task_indices, fused_table):
    B = task_indices.shape[0]
    return pl.pallas_call(
        _gather_direct_kernel,
        out_shape=jax.ShapeDtypeStruct((B, fused_table.shape[1]), fused_table.dtype),
        in_specs=[
            pl.BlockSpec(memory_space=pltpu.MemorySpace.SMEM),
            pl.BlockSpec(memory_space=pl.ANY),
        ],
        out_specs=pl.BlockSpec(memory_space=pl.ANY),
        scratch_shapes=[pltpu.SemaphoreType.DMA],
    )(task_indices.astype(jnp.int32), fused_table)


# speedup vs baseline: 1.3340x; 1.3340x over previous
import jax
import jax.numpy as jnp
from jax.experimental import pallas as pl
from jax.experimental.pallas import tpu as pltpu


def _gather_direct_kernel(idx_ref, lut_ref, out_ref, sem):
    B = out_ref.shape[0]
    for b in range(B):
        pltpu.make_async_copy(
            lut_ref.at[pl.ds(idx_ref[b], 1), :],
            out_ref.at[pl.ds(b, 1), :],
            sem,
        ).start()
    pltpu.make_async_copy(
        lut_ref.at[pl.ds(0, B), :],
        out_ref.at[pl.ds(0, B), :],
        sem,
    ).wait()


def kernel(task_indices, fused_table):
    B = task_indices.shape[0]
    return pl.pallas_call(
        _gather_direct_kernel,
        out_shape=jax.ShapeDtypeStruct((B, fused_table.shape[1]), fused_table.dtype),
        in_specs=[
            pl.BlockSpec(memory_space=pltpu.MemorySpace.SMEM),
            pl.BlockSpec(memory_space=pl.ANY),
        ],
        out_specs=pl.BlockSpec(memory_space=pltpu.MemorySpace.VMEM),
        scratch_shapes=[pltpu.SemaphoreType.DMA],
    )(task_indices.astype(jnp.int32), fused_table)
